# Initial kernel scaffold; baseline (speedup 1.0000x reference)
import functools

import jax
import jax.numpy as jnp
from jax import lax
from jax.experimental import pallas as pl
from jax.experimental.pallas import tpu as pltpu
from jax.experimental.pallas import tpu_sc as plsc

N = 10000
E = 320000
B = 16
NF = 128
EF = 16
GF = 32
H = 32
NP = 3
EOUT = 4

F32 = jnp.float32

NC = 2
NS = 16
NW = NC * NS
EPW = E // NW
C = 80
NCH = EPW // C
STR = N // NS

_MESH = plsc.VectorSubcoreMesh(
    core_axis_name="c", subcore_axis_name="s", num_cores=NC, num_subcores=NS)


def _lrelu(t):
    return jnp.maximum(t, 0.01 * t)


@functools.partial(
    pl.kernel,
    out_type=jax.ShapeDtypeStruct((E, H), F32),
    mesh=_MESH,
    scratch_types=[
        pltpu.VMEM_SHARED((N, H), F32),
        pltpu.VMEM_SHARED((N, H), F32),
        pltpu.VMEM((STR, H), F32),
        pltpu.VMEM((C,), jnp.int32),
        pltpu.VMEM((C,), jnp.int32),
        pltpu.VMEM((C, H), F32),
        pltpu.VMEM((C, H), F32),
        pltpu.SemaphoreType.DMA,
        pltpu.SemaphoreType.DMA,
    ],
)
def _sc_gather(row_hbm, col_hbm, xa_hbm, xb_hbm, g_hbm,
               xa_sh, xb_sh, stage, idxa, idxb, bufa, bufb, sema, semb):
    cid = lax.axis_index("c")
    sid = lax.axis_index("s")
    wid = sid * NC + cid
    base = sid * STR
    pltpu.sync_copy(xa_hbm.at[pl.ds(base, STR)], stage)
    pltpu.sync_copy(stage, xa_sh.at[pl.ds(base, STR)])
    pltpu.sync_copy(xb_hbm.at[pl.ds(base, STR)], stage)
    pltpu.sync_copy(stage, xb_sh.at[pl.ds(base, STR)])
    plsc.subcore_barrier()

    ebase = wid * EPW

    def chunk(j, carry):
        off = ebase + j * C
        pltpu.sync_copy(row_hbm.at[pl.ds(off, C)], idxa)
        pltpu.sync_copy(col_hbm.at[pl.ds(off, C)], idxb)
        cpa = pltpu.async_copy(xa_sh.at[idxa], bufa, sema)
        cpb = pltpu.async_copy(xb_sh.at[idxb], bufb, semb)
        cpa.wait()
        cpb.wait()

        def add_row(i, c2):
            bufa[i, pl.ds(0, 16)] = bufa[i, pl.ds(0, 16)] + bufb[i, pl.ds(0, 16)]
            bufa[i, pl.ds(16, 16)] = bufa[i, pl.ds(16, 16)] + bufb[i, pl.ds(16, 16)]
            return c2

        lax.fori_loop(0, C, add_row, 0)
        pltpu.sync_copy(bufa, g_hbm.at[pl.ds(off, C)])
        return carry

    lax.fori_loop(0, NCH, chunk, 0)


@functools.partial(
    pl.kernel,
    out_type=jax.ShapeDtypeStruct((NC, N, H), F32),
    mesh=_MESH,
    scratch_types=[
        pltpu.VMEM_SHARED((N, H), F32),
        pltpu.VMEM((STR, H), F32),
        pltpu.VMEM((C,), jnp.int32),
        pltpu.VMEM((C, H), F32),
    ],
)
def _sc_scatter(enew_hbm, col_hbm, agg_hbm, agg_sh, stage, idx, buf):
    cid = lax.axis_index("c")
    sid = lax.axis_index("s")
    wid = sid * NC + cid
    zero = jnp.zeros((16,), F32)

    def zrow(i, c2):
        stage[i, pl.ds(0, 16)] = zero
        stage[i, pl.ds(16, 16)] = zero
        return c2

    lax.fori_loop(0, STR, zrow, 0)
    pltpu.sync_copy(stage, agg_sh.at[pl.ds(sid * STR, STR)])
    plsc.subcore_barrier()

    ebase = wid * EPW

    def chunk(j, c2):
        off = ebase + j * C
        pltpu.sync_copy(col_hbm.at[pl.ds(off, C)], idx)
        pltpu.sync_copy(enew_hbm.at[pl.ds(off, C)], buf)
        pltpu.sync_copy(buf, agg_sh.at[idx], add=True)
        return c2

    lax.fori_loop(0, NCH, chunk, 0)
    plsc.subcore_barrier()
    pltpu.sync_copy(agg_sh.at[pl.ds(sid * STR, STR)], stage)
    pltpu.sync_copy(stage, agg_hbm.at[cid, pl.ds(sid * STR, STR)])


def _prep_body(x_ref, u_ref, b2d_ref, br_ref, wn_ref, bn_ref, wg_ref, bg_ref,
               w1a_ref, w1b_ref, w1d_ref,
               x1_ref, u1_ref, oh_ref, oht_ref, xa_ref, xb_ref):
    x1 = _lrelu(x_ref[...] @ wn_ref[...] + bn_ref[...])
    u1 = _lrelu(u_ref[...] @ wg_ref[...] + bg_ref[...])
    oh = (b2d_ref[...] == lax.broadcasted_iota(jnp.int32, (N, B), 1)).astype(F32)
    oht = (br_ref[...] == lax.broadcasted_iota(jnp.int32, (B, N), 0)).astype(F32)
    x1_ref[...] = x1
    u1_ref[...] = u1
    oh_ref[...] = oh
    oht_ref[...] = oht
    xa_ref[...] = x1 @ w1a_ref[...] + oh @ (u1 @ w1d_ref[...])
    xb_ref[...] = x1 @ w1b_ref[...]


def _edge_body(g_ref, a_ref, we_ref, be_ref, w1c_ref, b1_ref, w2_ref, b2_ref,
               w3_ref, b3_ref, dw_ref, db_ref, enew_ref, eout_ref,
               *, first, last):
    a = a_ref[...]
    if first:
        a = _lrelu(a @ we_ref[...] + be_ref[...])
    h = _lrelu(g_ref[...] + a @ w1c_ref[...] + b1_ref[...])
    h = _lrelu(h @ w2_ref[...] + b2_ref[...])
    en = h @ w3_ref[...] + b3_ref[...]
    enew_ref[...] = en
    if last:
        eout_ref[...] = (a + en) @ dw_ref[...] + db_ref[...]
    else:
        eout_ref[...] = a + en


def _node_body(x_ref, agg_ref, u_ref, oh_ref, oht_ref,
               na_ref, nb_ref, nc_ref, nb1_ref, nw2_ref, nb2_ref, nw3_ref,
               nb3_ref, ga_ref, gbw_ref, gb1_ref, gw2_ref, gb2_ref, gw3_ref,
               gb3_ref, wa_ref, wb_ref, wd_ref, *out_refs, last):
    x = x_ref[...]
    u = u_ref[...]
    oh = oh_ref[...]
    agg = agg_ref[0] + agg_ref[1]
    h = _lrelu(x @ na_ref[...] + agg @ nb_ref[...] + oh @ (u @ nc_ref[...])
               + nb1_ref[...])
    h = _lrelu(h @ nw2_ref[...] + nb2_ref[...])
    xn = h @ nw3_ref[...] + nb3_ref[...]
    x2 = x + xn
    sx = oht_ref[...] @ xn
    gh = _lrelu(u @ ga_ref[...] + sx @ gbw_ref[...] + gb1_ref[...])
    gh = _lrelu(gh @ gw2_ref[...] + gb2_ref[...])
    u2 = u + gh @ gw3_ref[...] + gb3_ref[...]
    if last:
        out_refs[0][...] = _lrelu(u2 @ wa_ref[...] + wd_ref[...]) @ wb_ref[...]
    else:
        out_refs[0][...] = x2
        out_refs[1][...] = u2
        out_refs[2][...] = x2 @ wa_ref[...] + oh @ (u2 @ wd_ref[...])
        out_refs[3][...] = x2 @ wb_ref[...]


BE = 8000


def _edge_call(g, a, we, be, w1c, b1, w2, b2, w3, b3, dw, db, first, last):
    af = a.shape[1]
    outf = EOUT if last else H

    def wspec(arr):
        shp = arr.shape
        return pl.BlockSpec(shp, lambda i: tuple(0 for _ in shp))

    return pl.pallas_call(
        functools.partial(_edge_body, first=first, last=last),
        grid=(E // BE,),
        in_specs=[
            pl.BlockSpec((BE, H), lambda i: (i, 0)),
            pl.BlockSpec((BE, af), lambda i: (i, 0)),
            wspec(we), wspec(be), wspec(w1c), wspec(b1),
            wspec(w2), wspec(b2), wspec(w3), wspec(b3),
            wspec(dw), wspec(db),
        ],
        out_specs=[
            pl.BlockSpec((BE, H), lambda i: (i, 0)),
            pl.BlockSpec((BE, outf), lambda i: (i, 0)),
        ],
        out_shape=[
            jax.ShapeDtypeStruct((E, H), F32),
            jax.ShapeDtypeStruct((E, outf), F32),
        ],
    )(g, a, we, be, w1c, b1, w2, b2, w3, b3, dw, db)


def kernel(x, edge_index, edge_attr, u, batch, params):
    p = params
    row = edge_index[0]
    col = edge_index[1]
    b2d = batch.reshape(N, 1)
    br = batch.reshape(1, N)

    def r2(b):
        return b.reshape(1, -1)

    ew1 = p['edge_W1']
    w1a = [ew1[i, 0:H] for i in range(NP)]
    w1b = [ew1[i, H:2 * H] for i in range(NP)]
    w1c = [ew1[i, 2 * H:3 * H] for i in range(NP)]
    w1d = [ew1[i, 3 * H:4 * H] for i in range(NP)]
    nw1 = p['node_W1']
    na = [nw1[i, 0:H] for i in range(NP)]
    nb = [nw1[i, H:2 * H] for i in range(NP)]
    ncw = [nw1[i, 2 * H:3 * H] for i in range(NP)]
    gw1 = p['glob_W1']
    ga = [gw1[i, 0:H] for i in range(NP)]
    gbw = [gw1[i, H:2 * H] for i in range(NP)]

    x1, u1, oh, oht, xa, xb = pl.pallas_call(
        _prep_body,
        out_shape=[
            jax.ShapeDtypeStruct((N, H), F32),
            jax.ShapeDtypeStruct((B, H), F32),
            jax.ShapeDtypeStruct((N, B), F32),
            jax.ShapeDtypeStruct((B, N), F32),
            jax.ShapeDtypeStruct((N, H), F32),
            jax.ShapeDtypeStruct((N, H), F32),
        ],
    )(x, u, b2d, br, p['emb_node_W'], r2(p['emb_node_b']),
      p['emb_glob_W'], r2(p['emb_glob_b']), w1a[0], w1b[0], w1d[0])

    e_attr = edge_attr
    edge_out = None
    value = None
    for i in range(NP):
        first = (i == 0)
        last = (i == NP - 1)
        g = _sc_gather(row, col, xa, xb)
        e_new, e_next = _edge_call(
            g, e_attr,
            p['emb_edge_W'], r2(p['emb_edge_b']),
            w1c[i], r2(p['edge_b1'][i]),
            p['edge_W2'][i], r2(p['edge_b2'][i]),
            p['edge_W3'][i], r2(p['edge_b3'][i]),
            p['dec_W'], r2(p['dec_b']),
            first, last)
        if last:
            edge_out = e_next
        else:
            e_attr = e_next
        agg2 = _sc_scatter(e_new, col)

        node_weights = (
            na[i], nb[i], ncw[i], r2(p['node_b1'][i]),
            p['node_W2'][i], r2(p['node_b2'][i]),
            p['node_W3'][i], r2(p['node_b3'][i]),
            ga[i], gbw[i], r2(p['glob_b1'][i]),
            p['glob_W2'][i], r2(p['glob_b2'][i]),
            p['glob_W3'][i], r2(p['glob_b3'][i]),
        )
        if last:
            value = pl.pallas_call(
                functools.partial(_node_body, last=True),
                out_shape=[jax.ShapeDtypeStruct((B, 1), F32)],
            )(x1, agg2, u1, oh, oht, *node_weights,
              p['val_W1'], p['val_W2'], r2(p['val_b1']))[0]
            value = value + p['val_b2'].reshape(1, 1)
        else:
            x1, u1, xa, xb = pl.pallas_call(
                functools.partial(_node_body, last=False),
                out_shape=[
                    jax.ShapeDtypeStruct((N, H), F32),
                    jax.ShapeDtypeStruct((B, H), F32),
                    jax.ShapeDtypeStruct((N, H), F32),
                    jax.ShapeDtypeStruct((N, H), F32),
                ],
            )(x1, agg2, u1, oh, oht, *node_weights,
              w1a[i + 1], w1b[i + 1], w1d[i + 1])

    return edge_out, value


# speedup vs baseline: 3.6145x; 3.6145x over previous
import functools

import jax
import jax.numpy as jnp
from jax import lax
from jax.experimental import pallas as pl
from jax.experimental.pallas import tpu as pltpu
from jax.experimental.pallas import tpu_sc as plsc

_HI = lax.Precision.HIGHEST

N = 10000
E = 320000
B = 16
NF = 128
EF = 16
GF = 32
H = 32
NP = 3
EOUT = 4

F32 = jnp.float32

NC = 2
NS = 16
NW = NC * NS
EPW = E // NW
C = 80
NCH = EPW // C
NPAD = 10240
STR = NPAD // NS

_MESH = plsc.VectorSubcoreMesh(
    core_axis_name="c", subcore_axis_name="s", num_cores=NC, num_subcores=NS)


def _lrelu(t):
    return jnp.maximum(t, 0.01 * t)


@functools.partial(
    pl.kernel,
    out_type=jax.ShapeDtypeStruct((E, H), F32),
    mesh=_MESH,
    compiler_params=pltpu.CompilerParams(use_tc_tiling_on_sc=False),
    scratch_types=[
        pltpu.VMEM((C,), jnp.int32),
        pltpu.VMEM((C,), jnp.int32),
        pltpu.VMEM((C, H), F32),
        pltpu.VMEM((C, H), F32),
        pltpu.SemaphoreType.DMA,
        pltpu.SemaphoreType.DMA,
    ],
)
def _sc_gather(row_hbm, col_hbm, xa_hbm, xb_hbm, g_hbm,
               idxa, idxb, bufa, bufb, sema, semb):
    cid = lax.axis_index("c")
    sid = lax.axis_index("s")
    wid = sid * NC + cid
    ebase = wid * EPW

    def chunk(j, carry):
        off = pl.multiple_of(ebase + j * C, 8)
        pltpu.sync_copy(row_hbm.at[pl.ds(off, C)], idxa)
        pltpu.sync_copy(col_hbm.at[pl.ds(off, C)], idxb)
        cpa = pltpu.async_copy(xa_hbm.at[idxa], bufa, sema)
        cpb = pltpu.async_copy(xb_hbm.at[idxb], bufb, semb)
        cpa.wait()
        cpb.wait()

        def add_row(i, c2):
            bufa[i, pl.ds(0, 16)] = bufa[i, pl.ds(0, 16)] + bufb[i, pl.ds(0, 16)]
            bufa[i, pl.ds(16, 16)] = bufa[i, pl.ds(16, 16)] + bufb[i, pl.ds(16, 16)]
            return c2

        lax.fori_loop(0, C, add_row, 0)
        pltpu.sync_copy(bufa, g_hbm.at[pl.ds(off, C)])
        return carry

    lax.fori_loop(0, NCH, chunk, 0)


@functools.partial(
    pl.kernel,
    out_type=jax.ShapeDtypeStruct((NC, NPAD, H), F32),
    mesh=_MESH,
    compiler_params=pltpu.CompilerParams(use_tc_tiling_on_sc=False),
    scratch_types=[
        pltpu.VMEM_SHARED((NPAD, H), F32),
        pltpu.VMEM((C, H), F32),
        pltpu.VMEM((C,), jnp.int32),
        pltpu.VMEM((C, H), F32),
    ],
)
def _sc_scatter(enew_hbm, col_hbm, agg_hbm, agg_sh, stage, idx, buf):
    cid = lax.axis_index("c")
    sid = lax.axis_index("s")
    wid = sid * NC + cid
    zero = jnp.zeros((16,), F32)

    def zrow(i, c2):
        stage[i, pl.ds(0, 16)] = zero
        stage[i, pl.ds(16, 16)] = zero
        return c2

    lax.fori_loop(0, C, zrow, 0)
    sbase = pl.multiple_of(sid * STR, 8)
    for k in range(STR // C):
        pltpu.sync_copy(stage, agg_sh.at[pl.ds(sbase + k * C, C)])
    plsc.subcore_barrier()

    ebase = wid * EPW

    def chunk(j, c2):
        off = pl.multiple_of(ebase + j * C, 8)
        pltpu.sync_copy(col_hbm.at[pl.ds(off, C)], idx)
        pltpu.sync_copy(enew_hbm.at[pl.ds(off, C)], buf)
        pltpu.sync_copy(buf, agg_sh.at[idx], add=True)
        return c2

    lax.fori_loop(0, NCH, chunk, 0)
    plsc.subcore_barrier()
    for k in range(STR // C):
        pltpu.sync_copy(agg_sh.at[pl.ds(sbase + k * C, C)], stage)
        pltpu.sync_copy(stage, agg_hbm.at[cid, pl.ds(sbase + k * C, C)])


BN = 2000
BE = 2000


def _oh(b2d_ref, rows):
    return (b2d_ref[...] ==
            lax.broadcasted_iota(jnp.int32, (rows, B), 1)).astype(F32)


def _prep_body(x_ref, u_ref, b2d_ref, wn_ref, bn_ref, wg_ref, bg_ref,
               w1a_ref, w1b_ref, w1d_ref,
               x1_ref, u1_ref, xa_ref, xb_ref):
    x1 = _lrelu(x_ref[...] @ wn_ref[...] + bn_ref[...])
    u1 = _lrelu(u_ref[...] @ wg_ref[...] + bg_ref[...])
    oh = _oh(b2d_ref, BN)
    x1_ref[...] = x1
    u1_ref[...] = u1
    xa_ref[...] = x1 @ w1a_ref[...] + jnp.dot(
        oh, u1 @ w1d_ref[...], precision=_HI)
    xb_ref[...] = x1 @ w1b_ref[...]


def _edge_body(g_ref, a_ref, we_ref, be_ref, w1c_ref, b1_ref, w2_ref, b2_ref,
               w3_ref, b3_ref, dw_ref, db_ref, enew_ref, eout_ref,
               *, first, last):
    a = a_ref[...]
    if first:
        a = _lrelu(a @ we_ref[...] + be_ref[...])
    h = _lrelu(g_ref[...] + a @ w1c_ref[...] + b1_ref[...])
    h = _lrelu(h @ w2_ref[...] + b2_ref[...])
    en = h @ w3_ref[...] + b3_ref[...]
    enew_ref[...] = en
    if last:
        eout_ref[...] = (a + en) @ dw_ref[...] + db_ref[...]
    else:
        eout_ref[...] = a + en


def _node_body(x_ref, agg_ref, b2d_ref, u_ref,
               na_ref, nb_ref, nc_ref, nb1_ref, nw2_ref, nb2_ref, nw3_ref,
               nb3_ref, x2_ref, xn_ref):
    x = x_ref[...]
    agg = agg_ref[0] + agg_ref[1]
    oh = _oh(b2d_ref, BN)
    h = _lrelu(x @ na_ref[...] + agg @ nb_ref[...]
               + jnp.dot(oh, u_ref[...] @ nc_ref[...], precision=_HI)
               + nb1_ref[...])
    h = _lrelu(h @ nw2_ref[...] + nb2_ref[...])
    xn = h @ nw3_ref[...] + nb3_ref[...]
    xn_ref[...] = xn
    x2_ref[...] = x + xn


def _glob_body(xn_ref, br_ref, u_ref, ga_ref, gbw_ref, gb1_ref, gw2_ref,
               gb2_ref, gw3_ref, gb3_ref, vw1_ref, vb1_ref, vw2_ref,
               *out_refs, last):
    u = u_ref[...]
    oht = (br_ref[...] ==
           lax.broadcasted_iota(jnp.int32, (B, N), 0)).astype(F32)
    sx = jnp.dot(oht, xn_ref[...], precision=_HI)
    gh = _lrelu(u @ ga_ref[...] + sx @ gbw_ref[...] + gb1_ref[...])
    gh = _lrelu(gh @ gw2_ref[...] + gb2_ref[...])
    u2 = u + gh @ gw3_ref[...] + gb3_ref[...]
    if last:
        out_refs[0][...] = _lrelu(u2 @ vw1_ref[...] + vb1_ref[...]) @ vw2_ref[...]
    else:
        out_refs[0][...] = u2


def _table_body(x2_ref, b2d_ref, u2_ref, w1a_ref, w1b_ref, w1d_ref,
                xa_ref, xb_ref):
    x2 = x2_ref[...]
    oh = _oh(b2d_ref, BN)
    xa_ref[...] = x2 @ w1a_ref[...] + jnp.dot(
        oh, u2_ref[...] @ w1d_ref[...], precision=_HI)
    xb_ref[...] = x2 @ w1b_ref[...]


def _bcast(arr):
    shp = arr.shape
    return pl.BlockSpec(shp, lambda i: tuple(0 for _ in shp))


def _edge_call(g, a, we, be, w1c, b1, w2, b2, w3, b3, dw, db, first, last):
    af = a.shape[1]
    outf = EOUT if last else H
    return pl.pallas_call(
        functools.partial(_edge_body, first=first, last=last),
        grid=(E // BE,),
        in_specs=[
            pl.BlockSpec((BE, H), lambda i: (i, 0)),
            pl.BlockSpec((BE, af), lambda i: (i, 0)),
            _bcast(we), _bcast(be), _bcast(w1c), _bcast(b1),
            _bcast(w2), _bcast(b2), _bcast(w3), _bcast(b3),
            _bcast(dw), _bcast(db),
        ],
        out_specs=[
            pl.BlockSpec((BE, H), lambda i: (i, 0)),
            pl.BlockSpec((BE, outf), lambda i: (i, 0)),
        ],
        out_shape=[
            jax.ShapeDtypeStruct((E, H), F32),
            jax.ShapeDtypeStruct((E, outf), F32),
        ],
    )(g, a, we, be, w1c, b1, w2, b2, w3, b3, dw, db)


def kernel(x, edge_index, edge_attr, u, batch, params):
    p = params
    row = edge_index[0]
    col = edge_index[1]
    b2d = batch.reshape(N, 1)
    br = batch.reshape(1, N)

    def r2(b):
        return b.reshape(1, -1)

    ew1 = p['edge_W1']
    w1a = [ew1[i, 0:H] for i in range(NP)]
    w1b = [ew1[i, H:2 * H] for i in range(NP)]
    w1c = [ew1[i, 2 * H:3 * H] for i in range(NP)]
    w1d = [ew1[i, 3 * H:4 * H] for i in range(NP)]
    nw1 = p['node_W1']
    na = [nw1[i, 0:H] for i in range(NP)]
    nb = [nw1[i, H:2 * H] for i in range(NP)]
    ncw = [nw1[i, 2 * H:3 * H] for i in range(NP)]
    gw1 = p['glob_W1']
    ga = [gw1[i, 0:H] for i in range(NP)]
    gbw = [gw1[i, H:2 * H] for i in range(NP)]

    nspec = pl.BlockSpec((BN, H), lambda i: (i, 0))
    bspec = pl.BlockSpec((BN, 1), lambda i: (i, 0))
    nshape = jax.ShapeDtypeStruct((N, H), F32)

    x1, u1, xa, xb = pl.pallas_call(
        _prep_body,
        grid=(N // BN,),
        in_specs=[
            pl.BlockSpec((BN, NF), lambda i: (i, 0)),
            _bcast(u), bspec,
            _bcast(p['emb_node_W']), pl.BlockSpec((1, H), lambda i: (0, 0)),
            _bcast(p['emb_glob_W']), pl.BlockSpec((1, H), lambda i: (0, 0)),
            _bcast(w1a[0]), _bcast(w1b[0]), _bcast(w1d[0]),
        ],
        out_specs=[nspec, pl.BlockSpec((B, H), lambda i: (0, 0)),
                   nspec, nspec],
        out_shape=[nshape, jax.ShapeDtypeStruct((B, H), F32), nshape, nshape],
    )(x, u, b2d, p['emb_node_W'], r2(p['emb_node_b']),
      p['emb_glob_W'], r2(p['emb_glob_b']), w1a[0], w1b[0], w1d[0])

    e_attr = edge_attr
    edge_out = None
    value = None
    for i in range(NP):
        first = (i == 0)
        last = (i == NP - 1)
        g = _sc_gather(row, col, xa, xb)
        e_new, e_next = _edge_call(
            g, e_attr,
            p['emb_edge_W'], r2(p['emb_edge_b']),
            w1c[i], r2(p['edge_b1'][i]),
            p['edge_W2'][i], r2(p['edge_b2'][i]),
            p['edge_W3'][i], r2(p['edge_b3'][i]),
            p['dec_W'], r2(p['dec_b']),
            first, last)
        if last:
            edge_out = e_next
        else:
            e_attr = e_next
        agg2 = _sc_scatter(e_new, col)

        x2, xn = pl.pallas_call(
            _node_body,
            grid=(N // BN,),
            in_specs=[
                nspec,
                pl.BlockSpec((NC, BN, H), lambda i: (0, i, 0)),
                bspec, _bcast(u1),
                _bcast(na[i]), _bcast(nb[i]), _bcast(ncw[i]),
                pl.BlockSpec((1, H), lambda i: (0, 0)),
                _bcast(p['node_W2'][i]), pl.BlockSpec((1, H), lambda i: (0, 0)),
                _bcast(p['node_W3'][i]), pl.BlockSpec((1, H), lambda i: (0, 0)),
            ],
            out_specs=[nspec, nspec],
            out_shape=[nshape, nshape],
        )(x1, agg2, b2d, u1, na[i], nb[i], ncw[i], r2(p['node_b1'][i]),
          p['node_W2'][i], r2(p['node_b2'][i]),
          p['node_W3'][i], r2(p['node_b3'][i]))

        glob_out_shape = (jax.ShapeDtypeStruct((B, 1), F32) if last
                          else jax.ShapeDtypeStruct((B, H), F32))
        gout = pl.pallas_call(
            functools.partial(_glob_body, last=last),
            out_shape=[glob_out_shape],
        )(xn, br, u1, ga[i], gbw[i], r2(p['glob_b1'][i]),
          p['glob_W2'][i], r2(p['glob_b2'][i]),
          p['glob_W3'][i], r2(p['glob_b3'][i]),
          p['val_W1'], r2(p['val_b1']), p['val_W2'])[0]

        if last:
            value = gout + p['val_b2'].reshape(1, 1)
        else:
            u2 = gout
            xa, xb = pl.pallas_call(
                _table_body,
                grid=(N // BN,),
                in_specs=[nspec, bspec, _bcast(u2),
                          _bcast(w1a[i + 1]), _bcast(w1b[i + 1]),
                          _bcast(w1d[i + 1])],
                out_specs=[nspec, nspec],
                out_shape=[nshape, nshape],
            )(x2, b2d, u2, w1a[i + 1], w1b[i + 1], w1d[i + 1])
            x1, u1 = x2, u2

    return edge_out, value


# speedup vs baseline: 4.6195x; 1.2781x over previous
import functools

import jax
import jax.numpy as jnp
from jax import lax
from jax.experimental import pallas as pl
from jax.experimental.pallas import tpu as pltpu
from jax.experimental.pallas import tpu_sc as plsc

_HI = lax.Precision.HIGHEST

N = 10000
E = 320000
B = 16
NF = 128
EF = 16
GF = 32
H = 32
NP = 3
EOUT = 4

F32 = jnp.float32

NC = 2
NS = 16
NW = NC * NS
EPW = E // NW
C = 80
NCH = EPW // C
NPAD = 10240
STR = NPAD // NS

_MESH = plsc.VectorSubcoreMesh(
    core_axis_name="c", subcore_axis_name="s", num_cores=NC, num_subcores=NS)


def _lrelu(t):
    return jnp.maximum(t, 0.01 * t)


@functools.partial(
    pl.kernel,
    out_type=jax.ShapeDtypeStruct((E, H), F32),
    mesh=_MESH,
    compiler_params=pltpu.CompilerParams(use_tc_tiling_on_sc=False),
    scratch_types=[
        pltpu.VMEM((2, C), jnp.int32),
        pltpu.VMEM((2, C), jnp.int32),
        pltpu.VMEM((2, C, H), F32),
        pltpu.VMEM((2, C, H), F32),
        pltpu.VMEM((2, C, H), F32),
        pltpu.SemaphoreType.DMA,
        pltpu.SemaphoreType.DMA,
        pltpu.SemaphoreType.DMA,
        pltpu.SemaphoreType.DMA,
        pltpu.SemaphoreType.DMA,
        pltpu.SemaphoreType.DMA,
    ],
)
def _sc_gather(row_hbm, col_hbm, xa_hbm, xb_hbm, g_hbm,
               idxa, idxb, bufa, bufb, obuf,
               sga0, sga1, sgb0, sgb1, sw0, sw1):
    cid = lax.axis_index("c")
    sid = lax.axis_index("s")
    wid = sid * NC + cid
    ebase = wid * EPW
    sga = (sga0, sga1)
    sgb = (sgb0, sgb1)
    sw = (sw0, sw1)

    def fire(j, b):
        off = pl.multiple_of(ebase + j * C, 8)
        pltpu.sync_copy(row_hbm.at[pl.ds(off, C)], idxa.at[b])
        pltpu.sync_copy(col_hbm.at[pl.ds(off, C)], idxb.at[b])
        pltpu.async_copy(xa_hbm.at[idxa.at[b]], bufa.at[b], sga[b])
        pltpu.async_copy(xb_hbm.at[idxb.at[b]], bufb.at[b], sgb[b])

    fire(0, 0)
    fire(1, 1)

    def body(t, carry):
        for b in range(2):
            j = 2 * t + b

            @pl.when(j < NCH)
            def _():
                off = pl.multiple_of(ebase + j * C, 8)
                pltpu.make_async_copy(
                    xa_hbm.at[idxa.at[b]], bufa.at[b], sga[b]).wait()
                pltpu.make_async_copy(
                    xb_hbm.at[idxb.at[b]], bufb.at[b], sgb[b]).wait()

                @pl.when(j >= 2)
                def _():
                    offp = pl.multiple_of(ebase + (j - 2) * C, 8)
                    pltpu.make_async_copy(
                        obuf.at[b], g_hbm.at[pl.ds(offp, C)], sw[b]).wait()

                def add_row(i, c2):
                    obuf[b, i, pl.ds(0, 16)] = (
                        bufa[b, i, pl.ds(0, 16)] + bufb[b, i, pl.ds(0, 16)])
                    obuf[b, i, pl.ds(16, 16)] = (
                        bufa[b, i, pl.ds(16, 16)] + bufb[b, i, pl.ds(16, 16)])
                    return c2

                lax.fori_loop(0, C, add_row, 0)
                pltpu.async_copy(obuf.at[b], g_hbm.at[pl.ds(off, C)], sw[b])

                @pl.when(j + 2 < NCH)
                def _():
                    fire(j + 2, b)
        return carry

    lax.fori_loop(0, (NCH + 2) // 2, body, 0)
    off_l0 = pl.multiple_of(ebase + (NCH - 1) * C, 8)
    off_l1 = pl.multiple_of(ebase + (NCH - 2) * C, 8)
    pltpu.make_async_copy(
        obuf.at[(NCH - 1) % 2], g_hbm.at[pl.ds(off_l0, C)],
        sw[(NCH - 1) % 2]).wait()
    pltpu.make_async_copy(
        obuf.at[(NCH - 2) % 2], g_hbm.at[pl.ds(off_l1, C)],
        sw[(NCH - 2) % 2]).wait()


@functools.partial(
    pl.kernel,
    out_type=jax.ShapeDtypeStruct((NC, NPAD, H), F32),
    mesh=_MESH,
    compiler_params=pltpu.CompilerParams(use_tc_tiling_on_sc=False),
    scratch_types=[
        pltpu.VMEM_SHARED((NPAD, H), F32),
        pltpu.VMEM((C, H), F32),
        pltpu.VMEM((2, C), jnp.int32),
        pltpu.VMEM((2, C, H), F32),
        pltpu.SemaphoreType.DMA,
        pltpu.SemaphoreType.DMA,
        pltpu.SemaphoreType.DMA,
        pltpu.SemaphoreType.DMA,
    ],
)
def _sc_scatter(enew_hbm, col_hbm, agg_hbm, agg_sh, stage, idx, buf,
                si0, si1, sd0, sd1):
    cid = lax.axis_index("c")
    sid = lax.axis_index("s")
    wid = sid * NC + cid
    zero = jnp.zeros((16,), F32)
    si = (si0, si1)
    sd = (sd0, sd1)
    ebase = wid * EPW

    def fire(j, b):
        off = pl.multiple_of(ebase + j * C, 8)
        pltpu.async_copy(col_hbm.at[pl.ds(off, C)], idx.at[b], si[b])
        pltpu.async_copy(enew_hbm.at[pl.ds(off, C)], buf.at[b], sd[b])

    fire(0, 0)
    fire(1, 1)

    def zrow(i, c2):
        stage[i, pl.ds(0, 16)] = zero
        stage[i, pl.ds(16, 16)] = zero
        return c2

    lax.fori_loop(0, C, zrow, 0)
    sbase = pl.multiple_of(sid * STR, 8)
    for k in range(STR // C):
        pltpu.sync_copy(stage, agg_sh.at[pl.ds(sbase + k * C, C)])
    plsc.subcore_barrier()

    def body(t, c2):
        for b in range(2):
            j = 2 * t + b

            @pl.when(j < NCH)
            def _():
                off = pl.multiple_of(ebase + j * C, 8)
                pltpu.make_async_copy(
                    col_hbm.at[pl.ds(off, C)], idx.at[b], si[b]).wait()
                pltpu.make_async_copy(
                    enew_hbm.at[pl.ds(off, C)], buf.at[b], sd[b]).wait()
                pltpu.sync_copy(buf.at[b], agg_sh.at[idx.at[b]], add=True)

                @pl.when(j + 2 < NCH)
                def _():
                    fire(j + 2, b)
        return c2

    lax.fori_loop(0, (NCH + 2) // 2, body, 0)
    plsc.subcore_barrier()
    for k in range(STR // C):
        pltpu.sync_copy(agg_sh.at[pl.ds(sbase + k * C, C)], stage)
        pltpu.sync_copy(stage, agg_hbm.at[cid, pl.ds(sbase + k * C, C)])


BN = 2000
BE = 2000


def _oh(b2d_ref, rows):
    return (b2d_ref[...] ==
            lax.broadcasted_iota(jnp.int32, (rows, B), 1)).astype(F32)


def _prep_body(x_ref, u_ref, b2d_ref, wn_ref, bn_ref, wg_ref, bg_ref,
               w1a_ref, w1b_ref, w1d_ref,
               x1_ref, u1_ref, xa_ref, xb_ref):
    x1 = _lrelu(x_ref[...] @ wn_ref[...] + bn_ref[...])
    u1 = _lrelu(u_ref[...] @ wg_ref[...] + bg_ref[...])
    oh = _oh(b2d_ref, BN)
    x1_ref[...] = x1
    u1_ref[...] = u1
    xa_ref[...] = x1 @ w1a_ref[...] + jnp.dot(
        oh, u1 @ w1d_ref[...], precision=_HI)
    xb_ref[...] = x1 @ w1b_ref[...]


def _edge_body(g_ref, a_ref, we_ref, be_ref, w1c_ref, b1_ref, w2_ref, b2_ref,
               w3_ref, b3_ref, dw_ref, db_ref, enew_ref, eout_ref,
               *, first, last):
    a = a_ref[...]
    if first:
        a = _lrelu(a @ we_ref[...] + be_ref[...])
    h = _lrelu(g_ref[...] + a @ w1c_ref[...] + b1_ref[...])
    h = _lrelu(h @ w2_ref[...] + b2_ref[...])
    en = h @ w3_ref[...] + b3_ref[...]
    enew_ref[...] = en
    if last:
        eout_ref[...] = (a + en) @ dw_ref[...] + db_ref[...]
    else:
        eout_ref[...] = a + en


def _node_body(x_ref, agg_ref, b2d_ref, u_ref,
               na_ref, nb_ref, nc_ref, nb1_ref, nw2_ref, nb2_ref, nw3_ref,
               nb3_ref, x2_ref, xn_ref):
    x = x_ref[...]
    agg = agg_ref[0] + agg_ref[1]
    oh = _oh(b2d_ref, BN)
    h = _lrelu(x @ na_ref[...] + agg @ nb_ref[...]
               + jnp.dot(oh, u_ref[...] @ nc_ref[...], precision=_HI)
               + nb1_ref[...])
    h = _lrelu(h @ nw2_ref[...] + nb2_ref[...])
    xn = h @ nw3_ref[...] + nb3_ref[...]
    xn_ref[...] = xn
    x2_ref[...] = x + xn


def _glob_body(xn_ref, br_ref, u_ref, ga_ref, gbw_ref, gb1_ref, gw2_ref,
               gb2_ref, gw3_ref, gb3_ref, vw1_ref, vb1_ref, vw2_ref,
               *out_refs, last):
    u = u_ref[...]
    oht = (br_ref[...] ==
           lax.broadcasted_iota(jnp.int32, (B, N), 0)).astype(F32)
    sx = jnp.dot(oht, xn_ref[...], precision=_HI)
    gh = _lrelu(u @ ga_ref[...] + sx @ gbw_ref[...] + gb1_ref[...])
    gh = _lrelu(gh @ gw2_ref[...] + gb2_ref[...])
    u2 = u + gh @ gw3_ref[...] + gb3_ref[...]
    if last:
        out_refs[0][...] = _lrelu(u2 @ vw1_ref[...] + vb1_ref[...]) @ vw2_ref[...]
    else:
        out_refs[0][...] = u2


def _table_body(x2_ref, b2d_ref, u2_ref, w1a_ref, w1b_ref, w1d_ref,
                xa_ref, xb_ref):
    x2 = x2_ref[...]
    oh = _oh(b2d_ref, BN)
    xa_ref[...] = x2 @ w1a_ref[...] + jnp.dot(
        oh, u2_ref[...] @ w1d_ref[...], precision=_HI)
    xb_ref[...] = x2 @ w1b_ref[...]


def _bcast(arr):
    shp = arr.shape
    return pl.BlockSpec(shp, lambda i: tuple(0 for _ in shp))


def _edge_call(g, a, we, be, w1c, b1, w2, b2, w3, b3, dw, db, first, last):
    af = a.shape[1]
    outf = EOUT if last else H
    return pl.pallas_call(
        functools.partial(_edge_body, first=first, last=last),
        grid=(E // BE,),
        in_specs=[
            pl.BlockSpec((BE, H), lambda i: (i, 0)),
            pl.BlockSpec((BE, af), lambda i: (i, 0)),
            _bcast(we), _bcast(be), _bcast(w1c), _bcast(b1),
            _bcast(w2), _bcast(b2), _bcast(w3), _bcast(b3),
            _bcast(dw), _bcast(db),
        ],
        out_specs=[
            pl.BlockSpec((BE, H), lambda i: (i, 0)),
            pl.BlockSpec((BE, outf), lambda i: (i, 0)),
        ],
        out_shape=[
            jax.ShapeDtypeStruct((E, H), F32),
            jax.ShapeDtypeStruct((E, outf), F32),
        ],
    )(g, a, we, be, w1c, b1, w2, b2, w3, b3, dw, db)


def kernel(x, edge_index, edge_attr, u, batch, params):
    p = params
    row = edge_index[0]
    col = edge_index[1]
    b2d = batch.reshape(N, 1)
    br = batch.reshape(1, N)

    def r2(b):
        return b.reshape(1, -1)

    ew1 = p['edge_W1']
    w1a = [ew1[i, 0:H] for i in range(NP)]
    w1b = [ew1[i, H:2 * H] for i in range(NP)]
    w1c = [ew1[i, 2 * H:3 * H] for i in range(NP)]
    w1d = [ew1[i, 3 * H:4 * H] for i in range(NP)]
    nw1 = p['node_W1']
    na = [nw1[i, 0:H] for i in range(NP)]
    nb = [nw1[i, H:2 * H] for i in range(NP)]
    ncw = [nw1[i, 2 * H:3 * H] for i in range(NP)]
    gw1 = p['glob_W1']
    ga = [gw1[i, 0:H] for i in range(NP)]
    gbw = [gw1[i, H:2 * H] for i in range(NP)]

    nspec = pl.BlockSpec((BN, H), lambda i: (i, 0))
    bspec = pl.BlockSpec((BN, 1), lambda i: (i, 0))
    nshape = jax.ShapeDtypeStruct((N, H), F32)

    x1, u1, xa, xb = pl.pallas_call(
        _prep_body,
        grid=(N // BN,),
        in_specs=[
            pl.BlockSpec((BN, NF), lambda i: (i, 0)),
            _bcast(u), bspec,
            _bcast(p['emb_node_W']), pl.BlockSpec((1, H), lambda i: (0, 0)),
            _bcast(p['emb_glob_W']), pl.BlockSpec((1, H), lambda i: (0, 0)),
            _bcast(w1a[0]), _bcast(w1b[0]), _bcast(w1d[0]),
        ],
        out_specs=[nspec, pl.BlockSpec((B, H), lambda i: (0, 0)),
                   nspec, nspec],
        out_shape=[nshape, jax.ShapeDtypeStruct((B, H), F32), nshape, nshape],
    )(x, u, b2d, p['emb_node_W'], r2(p['emb_node_b']),
      p['emb_glob_W'], r2(p['emb_glob_b']), w1a[0], w1b[0], w1d[0])

    e_attr = edge_attr
    edge_out = None
    value = None
    for i in range(NP):
        first = (i == 0)
        last = (i == NP - 1)
        g = _sc_gather(row, col, xa, xb)
        e_new, e_next = _edge_call(
            g, e_attr,
            p['emb_edge_W'], r2(p['emb_edge_b']),
            w1c[i], r2(p['edge_b1'][i]),
            p['edge_W2'][i], r2(p['edge_b2'][i]),
            p['edge_W3'][i], r2(p['edge_b3'][i]),
            p['dec_W'], r2(p['dec_b']),
            first, last)
        if last:
            edge_out = e_next
        else:
            e_attr = e_next
        agg2 = _sc_scatter(e_new, col)

        x2, xn = pl.pallas_call(
            _node_body,
            grid=(N // BN,),
            in_specs=[
                nspec,
                pl.BlockSpec((NC, BN, H), lambda i: (0, i, 0)),
                bspec, _bcast(u1),
                _bcast(na[i]), _bcast(nb[i]), _bcast(ncw[i]),
                pl.BlockSpec((1, H), lambda i: (0, 0)),
                _bcast(p['node_W2'][i]), pl.BlockSpec((1, H), lambda i: (0, 0)),
                _bcast(p['node_W3'][i]), pl.BlockSpec((1, H), lambda i: (0, 0)),
            ],
            out_specs=[nspec, nspec],
            out_shape=[nshape, nshape],
        )(x1, agg2, b2d, u1, na[i], nb[i], ncw[i], r2(p['node_b1'][i]),
          p['node_W2'][i], r2(p['node_b2'][i]),
          p['node_W3'][i], r2(p['node_b3'][i]))

        glob_out_shape = (jax.ShapeDtypeStruct((B, 1), F32) if last
                          else jax.ShapeDtypeStruct((B, H), F32))
        gout = pl.pallas_call(
            functools.partial(_glob_body, last=last),
            out_shape=[glob_out_shape],
        )(xn, br, u1, ga[i], gbw[i], r2(p['glob_b1'][i]),
          p['glob_W2'][i], r2(p['glob_b2'][i]),
          p['glob_W3'][i], r2(p['glob_b3'][i]),
          p['val_W1'], r2(p['val_b1']), p['val_W2'])[0]

        if last:
            value = gout + p['val_b2'].reshape(1, 1)
        else:
            u2 = gout
            xa, xb = pl.pallas_call(
                _table_body,
                grid=(N // BN,),
                in_specs=[nspec, bspec, _bcast(u2),
                          _bcast(w1a[i + 1]), _bcast(w1b[i + 1]),
                          _bcast(w1d[i + 1])],
                out_specs=[nspec, nspec],
                out_shape=[nshape, nshape],
            )(x2, b2d, u2, w1a[i + 1], w1b[i + 1], w1d[i + 1])
            x1, u1 = x2, u2

    return edge_out, value
